# baseline (speedup 1.0000x reference)
import jax
import jax.numpy as jnp
from jax import lax
from jax.experimental import pallas as pl
from jax.experimental.pallas import tpu as pltpu
from jax.experimental.pallas import tpu_sc as plsc

N_NODES = 10000
N_EDGES = 320000
HEADS = 8
ROW = 128

NC = 2
NS = 16
NW = NC * NS
EPW = N_EDGES // NW
CH = 32
NF = 312


def _leaky(x):
    return jnp.maximum(x, 0.2 * x)


def _make_edge_kernel(shift):
    mesh = plsc.VectorSubcoreMesh(
        core_axis_name="c", subcore_axis_name="s", num_cores=NC,
        num_subcores=NS)

    def body(tab, adp, eis, eid, out, acc, advmem, sbufa, si, wbuf,
             di0, di1, di2, di3, si_t, di_t,
             gsem0, gsem1, ssem0, ssem1, isem0, isem1, isem2, isem3):
        cid = lax.axis_index("c")
        sid = lax.axis_index("s")
        ebase = (cid * NS + sid) * EPW
        sbufs = [sbufa.at[pl.ds(0, CH)], sbufa.at[pl.ds(CH, CH)]]
        sidxs = [si.at[pl.ds(CH * k, CH)] for k in range(4)]
        didxs = [di0, di1, di2, di3]
        gsems = [gsem0, gsem1]
        ssems = [ssem0, ssem1]
        isems = [isem0, isem1, isem2, isem3]

        lanes = lax.iota(jnp.int32, 16)
        pats = []
        for edge in range(2):
            for g in range(5):
                c = lanes + g * 16
                sel = jnp.where(c < 64, c >> shift, (c - 64) & 7)
                pats.append(sel + edge * 8)
        himask = lanes >= 8
        ksel = (lanes >> 1) & 3
        even = (lanes & 1) == 0

        def issue_idx(off, ib):
            pltpu.async_copy(eis.at[pl.ds(off, CH)], sidxs[ib], isems[ib])
            pltpu.async_copy(eid.at[pl.ds(off, CH)], didxs[ib], isems[ib])

        def wait_idx(ib):
            pltpu.make_async_copy(eis.at[pl.ds(0, CH)], sidxs[ib],
                                  isems[ib]).wait()
            pltpu.make_async_copy(eid.at[pl.ds(0, CH)], didxs[ib],
                                  isems[ib]).wait()

        def issue_gather(sb, ib):
            pltpu.async_copy(tab.at[sidxs[ib]], sbufs[sb], gsems[sb])

        def wait_gather(sb, ib):
            pltpu.make_async_copy(tab.at[sidxs[ib]], sbufs[sb],
                                  gsems[sb]).wait()

        def issue_scatter(sb, ib):
            pltpu.async_copy(sbufs[sb], acc.at[didxs[ib]], ssems[sb],
                             add=True)

        def wait_scatter(sb, ib):
            pltpu.make_async_copy(sbufs[sb], acc.at[didxs[ib]],
                                  ssems[sb]).wait()

        def compute(sbuf, didx, n_edges):
            @plsc.parallel_loop(0, n_edges, 1, unroll=4)
            def wg(e):
                dv = plsc.load_gather(didx, [jnp.zeros((16,), jnp.int32)
                                             + e])
                wv = plsc.load_gather(advmem, [dv * 4 + ksel])
                lo = plsc.bitcast(wv << 16, jnp.float32)
                hi = plsc.bitcast(wv & -65536, jnp.float32)
                v = sbuf[e, pl.ds(64, 16)] + jnp.where(even, lo, hi)
                plsc.store_compressed(wbuf.at[pl.ds(e * 8, 16)],
                                      jnp.exp(_leaky(v)), mask=himask)

            @plsc.parallel_loop(0, n_edges // 2, 1, unroll=2)
            def mg(p):
                for edge in range(2):
                    for g in range(5):
                        v = sbuf[2 * p + edge, pl.ds(g * 16, 16)]
                        pw = plsc.load_gather(wbuf, [pats[edge * 5 + g]
                                                     + p * 16])
                        sbuf[2 * p + edge, pl.ds(g * 16, 16)] = v * pw

        pltpu.sync_copy(adp, advmem)
        issue_idx(ebase, 0)
        issue_idx(ebase + CH, 1)
        issue_idx(ebase + 2 * CH, 2)
        wait_idx(0)
        issue_gather(0, 0)

        def zrow(i, _):
            for g in range(8):
                sbufa[CH + i, pl.ds(g * 16, 16)] = jnp.zeros(
                    (16,), jnp.float32)
            return 0
        lax.fori_loop(0, CH, zrow, 0)

        @pl.when(sid < 10)
        def _():
            r0 = sid * 1000
            for k in range(31):
                pltpu.sync_copy(sbufs[1], acc.at[pl.ds(r0 + k * 32, 32)])
            pltpu.sync_copy(sbufa.at[pl.ds(CH, 8)],
                            acc.at[pl.ds(r0 + 992, 8)])
        plsc.subcore_barrier()

        def step(t, _):
            for q in range(4):
                sb = q & 1
                nsb = 1 - sb
                ib = q
                nib = (q + 1) & 3
                c_off = ebase + (4 * t + q) * CH
                wait_gather(sb, ib)
                if q == 0:
                    @pl.when(t > 0)
                    def _():
                        wait_scatter(nsb, (q - 1) & 3)
                else:
                    wait_scatter(nsb, (q - 1) & 3)
                if q == 3:
                    @pl.when(t < 77)
                    def _():
                        wait_idx(nib)
                        issue_gather(nsb, nib)
                else:
                    wait_idx(nib)
                    issue_gather(nsb, nib)
                if q == 0:
                    issue_idx(c_off + 3 * CH, (q + 3) & 3)
                else:
                    @pl.when(t < 77)
                    def _():
                        issue_idx(c_off + 3 * CH, (q + 3) & 3)
                compute(sbufs[sb], didxs[ib], CH)
                issue_scatter(sb, ib)
            return 0
        lax.fori_loop(0, 78, step, 0)
        wait_scatter(1, 3)

        offt = ebase + NF * CH
        pltpu.sync_copy(eis.at[pl.ds(offt, 16)], si_t)
        pltpu.sync_copy(eid.at[pl.ds(offt, 16)], di_t)
        pltpu.sync_copy(tab.at[si_t], sbufa.at[pl.ds(0, 16)])
        compute(sbufa.at[pl.ds(0, 16)], di_t, 16)
        pltpu.sync_copy(sbufa.at[pl.ds(0, 16)], acc.at[di_t], add=True)

        plsc.subcore_barrier()

        @pl.when(sid < 10)
        def _():
            r0 = sid * 1000
            ob = cid * N_NODES + r0
            for k in range(7):
                pltpu.sync_copy(acc.at[pl.ds(r0 + k * 128, 128)],
                                out.at[pl.ds(ob + k * 128, 128)])
            pltpu.sync_copy(acc.at[pl.ds(r0 + 896, 104)],
                            out.at[pl.ds(ob + 896, 104)])

    return pl.kernel(
        body,
        out_type=jax.ShapeDtypeStruct((2 * N_NODES, ROW), jnp.float32),
        mesh=mesh,
        compiler_params=pltpu.CompilerParams(needs_layout_passes=False),
        scratch_types=[
            pltpu.VMEM_SHARED((N_NODES, ROW), jnp.float32),
            pltpu.VMEM((N_NODES * 4,), jnp.int32),
            pltpu.VMEM((2 * CH, ROW), jnp.float32),
            pltpu.VMEM((4 * CH,), jnp.int32),
            pltpu.VMEM((CH * 8 + 8,), jnp.float32),
            pltpu.VMEM((CH,), jnp.int32),
            pltpu.VMEM((CH,), jnp.int32),
            pltpu.VMEM((CH,), jnp.int32),
            pltpu.VMEM((CH,), jnp.int32),
            pltpu.VMEM((16,), jnp.int32),
            pltpu.VMEM((16,), jnp.int32),
            pltpu.SemaphoreType.DMA,
            pltpu.SemaphoreType.DMA,
            pltpu.SemaphoreType.DMA,
            pltpu.SemaphoreType.DMA,
            pltpu.SemaphoreType.DMA,
            pltpu.SemaphoreType.DMA,
            pltpu.SemaphoreType.DMA,
            pltpu.SemaphoreType.DMA,
        ],
    )


_edge_l12 = _make_edge_kernel(3)
_edge_l3 = _make_edge_kernel(4)


_BLK = 1000
_GRID = N_NODES // _BLK


def _full(shape):
    return pl.BlockSpec(shape, lambda i: (0,) * len(shape))


def _rows(w):
    return pl.BlockSpec((_BLK, w), lambda i: (i, 0))


def _node_tail(h, As, Ad, E):
    a_s = jnp.dot(h, As, preferred_element_type=jnp.float32)
    a_d = jnp.dot(h, Ad, preferred_element_type=jnp.float32)
    w = jnp.exp(_leaky(a_s + a_d))
    wexp = jnp.dot(w, E, preferred_element_type=jnp.float32)
    ones = jnp.ones(a_s.shape, jnp.float32)
    z48 = jnp.zeros((a_s.shape[0], 48), jnp.float32)
    z56 = jnp.zeros((a_s.shape[0], 56), jnp.float32)
    table = jnp.concatenate([h, ones, a_s, z48], axis=1)
    selfmsg = jnp.concatenate([h * wexp, w, z56], axis=1)
    return table, a_d, selfmsg


def _stage1_body(x, W, As, Ad, E, table, adt, selfmsg):
    h = jnp.dot(x[...], W[...], preferred_element_type=jnp.float32)
    t, a, s = _node_tail(h, As[...], Ad[...], E[...])
    table[...], adt[...], selfmsg[...] = t, a, s


def _stage2_body(p0, p1, sm, x, Wx, Wh, b, As, Ad, E,
                 table, adt, selfmsg):
    tot = p0[...] + p1[...] + sm[...]
    r = 1.0 / (tot[:, 64:72] + 1e-16)
    t = tot[:, :64] * jnp.dot(r, E[...],
                              preferred_element_type=jnp.float32) + b[...]
    h1 = jnp.where(t > 0, t, jnp.exp(t) - 1.0)
    h = (jnp.dot(x[...], Wx[...], preferred_element_type=jnp.float32)
         + jnp.dot(h1, Wh[...], preferred_element_type=jnp.float32))
    t_, a_, s_ = _node_tail(h, As[...], Ad[...], E[...])
    table[...], adt[...], selfmsg[...] = t_, a_, s_


def _stage3_body(p0, p1, sm, W, b, As, Ad, E8, E16,
                 taba, tabb, adt, selfa, selfb):
    tot = p0[...] + p1[...] + sm[...]
    r = 1.0 / (tot[:, 64:72] + 1e-16)
    t = tot[:, :64] * jnp.dot(r, E8[...],
                              preferred_element_type=jnp.float32) + b[...]
    h2 = jnp.where(t > 0, t, jnp.exp(t) - 1.0)
    h3 = jnp.dot(h2, W[...], preferred_element_type=jnp.float32)
    a_s = jnp.dot(h3, As[...], preferred_element_type=jnp.float32)
    a_d = jnp.dot(h3, Ad[...], preferred_element_type=jnp.float32)
    w = jnp.exp(_leaky(a_s + a_d))
    wexp = jnp.dot(w, E16[...], preferred_element_type=jnp.float32)
    B = a_s.shape[0]
    ones = jnp.ones((B, 8), jnp.float32)
    z4 = jnp.zeros((B, 4), jnp.float32)
    z48 = jnp.zeros((B, 48), jnp.float32)
    z56 = jnp.zeros((B, 56), jnp.float32)
    hw = h3 * wexp
    taba[...] = jnp.concatenate([h3[:, :64], ones, a_s[:, :4], z4, z48], 1)
    tabb[...] = jnp.concatenate([h3[:, 64:], ones, a_s[:, 4:], z4, z48], 1)
    adt[...] = a_d
    selfa[...] = jnp.concatenate([hw[:, :64], w[:, :4], z4, z56], 1)
    selfb[...] = jnp.concatenate([hw[:, 64:], w[:, 4:], z4, z56], 1)


def _stage4_body(p0a, p1a, sma, p0b, p1b, smb, Ma, Mb, b, E4, out):
    tota = p0a[...] + p1a[...] + sma[...]
    totb = p0b[...] + p1b[...] + smb[...]
    ra = 1.0 / (tota[:, 64:68] + 1e-16)
    rb = 1.0 / (totb[:, 64:68] + 1e-16)
    va = tota[:, :64] * jnp.dot(ra, E4[...],
                                preferred_element_type=jnp.float32)
    vb = totb[:, :64] * jnp.dot(rb, E4[...],
                                preferred_element_type=jnp.float32)
    o = (jnp.dot(va, Ma[...], preferred_element_type=jnp.float32)
         + jnp.dot(vb, Mb[...], preferred_element_type=jnp.float32)
         + b[...])
    m = jnp.max(o, axis=1, keepdims=True)
    ls = jnp.log(jnp.sum(jnp.exp(o - m), axis=1, keepdims=True))
    out[...] = o - m - ls


def _pack_ad(a):
    b = a.astype(jnp.bfloat16).reshape(N_NODES, 4, 2)
    return jax.lax.bitcast_convert_type(b, jnp.int32).reshape(-1)


def _scatter_mat(a, ch):
    return jnp.einsum("hc,hk->hck", a, jnp.eye(HEADS, dtype=a.dtype)
                      ).reshape(HEADS * ch, HEADS)


def kernel(x, edge_index, W1, a1s, a1d, b1, W2, a2s, a2d, b2,
           W3, a3s, a3d, b3):
    ei = edge_index.astype(jnp.int32)
    eis, eid = ei[0], ei[1]
    E8 = jnp.repeat(jnp.eye(HEADS, dtype=jnp.float32), 8, axis=1)
    E16 = jnp.repeat(jnp.eye(HEADS, dtype=jnp.float32), 16, axis=1)
    E4 = jnp.repeat(jnp.eye(4, dtype=jnp.float32), 16, axis=1)
    Ma = jnp.tile(jnp.eye(16, dtype=jnp.float32), (4, 1)) / HEADS
    As1, Ad1 = _scatter_mat(a1s[0], 8), _scatter_mat(a1d[0], 8)
    As2, Ad2 = _scatter_mat(a2s[0], 8), _scatter_mat(a2d[0], 8)
    As3, Ad3 = _scatter_mat(a3s[0], 16), _scatter_mat(a3d[0], 16)

    f32 = jnp.float32
    sds = jax.ShapeDtypeStruct
    out3 = [sds((N_NODES, 128), f32), sds((N_NODES, 8), f32),
            sds((N_NODES, 128), f32)]

    tab1, ad1, self1 = pl.pallas_call(
        _stage1_body, grid=(_GRID,),
        in_specs=[_rows(128), _full((128, 64)), _full((64, 8)),
                  _full((64, 8)), _full((8, 64))],
        out_specs=[_rows(128), _rows(8), _rows(128)],
        out_shape=out3,
    )(x, W1, As1, Ad1, E8)

    part1 = _edge_l12(tab1, _pack_ad(ad1), eis, eid)

    tab2, ad2, self2 = pl.pallas_call(
        _stage2_body, grid=(_GRID,),
        in_specs=[_rows(128), _rows(128), _rows(128), _rows(128),
                  _full((128, 64)), _full((64, 64)), _full((1, 64)),
                  _full((64, 8)), _full((64, 8)), _full((8, 64))],
        out_specs=[_rows(128), _rows(8), _rows(128)],
        out_shape=out3,
    )(part1[:N_NODES], part1[N_NODES:], self1, x, W2[:128], W2[128:],
      b1.reshape(1, 64), As2, Ad2, E8)

    part2 = _edge_l12(tab2, _pack_ad(ad2), eis, eid)

    tab3a, tab3b, ad3, self3a, self3b = pl.pallas_call(
        _stage3_body, grid=(_GRID,),
        in_specs=[_rows(128), _rows(128), _rows(128),
                  _full((64, 128)), _full((1, 64)),
                  _full((128, 8)), _full((128, 8)), _full((8, 64)),
                  _full((8, 128))],
        out_specs=[_rows(128), _rows(128), _rows(8), _rows(128),
                   _rows(128)],
        out_shape=[sds((N_NODES, 128), f32), sds((N_NODES, 128), f32),
                   sds((N_NODES, 8), f32), sds((N_NODES, 128), f32),
                   sds((N_NODES, 128), f32)],
    )(part2[:N_NODES], part2[N_NODES:], self2, W3, b2.reshape(1, 64),
      As3, Ad3, E8, E16)

    part3a = _edge_l3(tab3a, _pack_ad(ad3), eis, eid)
    part3b = _edge_l3(tab3b, _pack_ad(jnp.concatenate(
        [ad3[:, 4:], ad3[:, :4]], axis=1)), eis, eid)

    out = pl.pallas_call(
        _stage4_body, grid=(_GRID,),
        in_specs=[_rows(128), _rows(128), _rows(128),
                  _rows(128), _rows(128), _rows(128),
                  _full((64, 16)), _full((64, 16)), _full((1, 16)),
                  _full((4, 64))],
        out_specs=_rows(16),
        out_shape=sds((N_NODES, 16), f32),
    )(part3a[:N_NODES], part3a[N_NODES:], self3a,
      part3b[:N_NODES], part3b[N_NODES:], self3b,
      Ma, Ma, b3.reshape(1, 16), E4)

    return out


# speedup vs baseline: 1.2182x; 1.2182x over previous
import jax
import jax.numpy as jnp
from jax import lax
from jax.experimental import pallas as pl
from jax.experimental.pallas import tpu as pltpu
from jax.experimental.pallas import tpu_sc as plsc

N_NODES = 10000
N_EDGES = 320000
HEADS = 8
ROW = 128

NC = 2
NS = 16
NW = NC * NS
EPW = N_EDGES // NW
CH = 64
NFULL = EPW // CH
TAIL = EPW - NFULL * CH


def _leaky(x):
    return jnp.maximum(x, 0.2 * x)


def _make_edge_kernel(shift):
    mesh = plsc.VectorSubcoreMesh(
        core_axis_name="c", subcore_axis_name="s", num_cores=NC,
        num_subcores=NS)

    def body(tab, adt, eis, eid, out, acc, sbuf0, sbuf1, abuf0, abuf1,
             wbuf, si0, si1, si2, si3, di0, di1, di2, di3,
             srcidx_t, dstidx_t, srcbuf_t, adbuf_t,
             gsem0, gsem1, ssem0, ssem1, isem0, isem1, isem2, isem3):
        cid = lax.axis_index("c")
        sid = lax.axis_index("s")
        ebase = (cid * NS + sid) * EPW
        sbufs = [sbuf0, sbuf1]
        abufs = [abuf0, abuf1]
        sidxs = [si0, si1, si2, si3]
        didxs = [di0, di1, di2, di3]
        gsems = [gsem0, gsem1]
        ssems = [ssem0, ssem1]
        isems = [isem0, isem1, isem2, isem3]

        lanes = lax.iota(jnp.int32, 16)
        pats = []
        for g in range(5):
            c = lanes + g * 16
            sel = jnp.where(c < 64, c >> shift, (c - 64) & 7)
            pats.append(sel + 8)

        def issue_idx(off, ib):
            pltpu.async_copy(eis.at[pl.ds(off, CH)], sidxs[ib], isems[ib])
            pltpu.async_copy(eid.at[pl.ds(off, CH)], didxs[ib], isems[ib])

        def wait_idx(ib):
            pltpu.make_async_copy(eis.at[pl.ds(0, CH)], sidxs[ib],
                                  isems[ib]).wait()
            pltpu.make_async_copy(eid.at[pl.ds(0, CH)], didxs[ib],
                                  isems[ib]).wait()

        def issue_gather(sb, ib):
            pltpu.async_copy(tab.at[sidxs[ib]], sbufs[sb], gsems[sb])
            pltpu.async_copy(adt.at[didxs[ib]], abufs[sb], gsems[sb])

        def wait_gather(sb, ib):
            pltpu.make_async_copy(tab.at[sidxs[ib]], sbufs[sb],
                                  gsems[sb]).wait()
            pltpu.make_async_copy(adt.at[didxs[ib]], abufs[sb],
                                  gsems[sb]).wait()

        def issue_scatter(sb, ib):
            pltpu.async_copy(sbufs[sb], acc.at[didxs[ib]], ssems[sb],
                             add=True)

        def wait_scatter(sb, ib):
            pltpu.make_async_copy(sbufs[sb], acc.at[didxs[ib]],
                                  ssems[sb]).wait()

        gdn = lax.GatherDimensionNumbers(
            offset_dims=(), collapsed_slice_dims=(0,), start_index_map=(0,))

        def vgather(v, idx):
            return lax.gather(v, idx[:, None], gdn, (1,),
                              mode=lax.GatherScatterMode.PROMISE_IN_BOUNDS)

        def compute(sbuf, abuf, n_edges):
            @plsc.parallel_loop(0, n_edges, 1, unroll=4)
            def wg(e):
                v = sbuf[e, pl.ds(64, 16)] + abuf[e, pl.ds(0, 16)]
                wbuf[pl.ds(e * 16, 16)] = jnp.exp(_leaky(v))

            @plsc.parallel_loop(0, n_edges // 2, 1, unroll=2)
            def mg(p):
                for edge in range(2):
                    wv = wbuf[pl.ds(p * 32 + edge * 16, 16)]
                    for g in range(5):
                        v = sbuf[2 * p + edge, pl.ds(g * 16, 16)]
                        pw = vgather(wv, pats[g])
                        sbuf[2 * p + edge, pl.ds(g * 16, 16)] = v * pw

        issue_idx(ebase, 0)
        issue_idx(ebase + CH, 1)
        issue_idx(ebase + 2 * CH, 2)
        wait_idx(0)
        issue_gather(0, 0)

        def zrow(i, _):
            for g in range(8):
                srcbuf_z = sbufs[1]
                srcbuf_z[i, pl.ds(g * 16, 16)] = jnp.zeros((16,),
                                                           jnp.float32)
            return 0
        lax.fori_loop(0, CH, zrow, 0)

        @pl.when(sid < 10)
        def _():
            r0 = sid * 1000
            for k in range(15):
                pltpu.sync_copy(sbufs[1], acc.at[pl.ds(r0 + k * 64, 64)])
            pltpu.sync_copy(sbufs[1].at[pl.ds(0, 40)],
                            acc.at[pl.ds(r0 + 960, 40)])
        plsc.subcore_barrier()

        def step(t, _):
            for q in range(4):
                sb = q & 1
                nsb = 1 - sb
                ib = q
                nib = (q + 1) & 3
                c_off = ebase + (4 * t + q) * CH
                wait_gather(sb, ib)
                if q == 0:
                    @pl.when(t > 0)
                    def _():
                        wait_scatter(nsb, (q - 1) & 3)
                else:
                    wait_scatter(nsb, (q - 1) & 3)
                if q == 3:
                    @pl.when(t < 38)
                    def _():
                        wait_idx(nib)
                        issue_gather(nsb, nib)
                else:
                    wait_idx(nib)
                    issue_gather(nsb, nib)
                if q == 0:
                    issue_idx(c_off + 3 * CH, (q + 3) & 3)
                else:
                    @pl.when(t < 38)
                    def _():
                        issue_idx(c_off + 3 * CH, (q + 3) & 3)
                compute(sbufs[sb], abufs[sb], CH)
                issue_scatter(sb, ib)
            return 0
        lax.fori_loop(0, 39, step, 0)
        wait_scatter(1, 3)

        offt = ebase + NFULL * CH
        pltpu.sync_copy(eis.at[pl.ds(offt, TAIL)], srcidx_t)
        pltpu.sync_copy(eid.at[pl.ds(offt, TAIL)], dstidx_t)
        pltpu.sync_copy(tab.at[srcidx_t], srcbuf_t)
        pltpu.sync_copy(adt.at[dstidx_t], adbuf_t)
        compute(srcbuf_t, adbuf_t, TAIL)
        pltpu.sync_copy(srcbuf_t, acc.at[dstidx_t], add=True)

        plsc.subcore_barrier()

        @pl.when(sid < 10)
        def _():
            r0 = sid * 1000
            ob = cid * N_NODES + r0
            for k in range(7):
                pltpu.sync_copy(acc.at[pl.ds(r0 + k * 128, 128)],
                                out.at[pl.ds(ob + k * 128, 128)])
            pltpu.sync_copy(acc.at[pl.ds(r0 + 896, 104)],
                            out.at[pl.ds(ob + 896, 104)])

    return pl.kernel(
        body,
        out_type=jax.ShapeDtypeStruct((2 * N_NODES, ROW), jnp.float32),
        mesh=mesh,
        compiler_params=pltpu.CompilerParams(needs_layout_passes=False),
        scratch_types=[
            pltpu.VMEM_SHARED((N_NODES, ROW), jnp.float32),
            pltpu.VMEM((CH, ROW), jnp.float32),
            pltpu.VMEM((CH, ROW), jnp.float32),
            pltpu.VMEM((CH, ROW), jnp.float32),
            pltpu.VMEM((CH, ROW), jnp.float32),
            pltpu.VMEM((CH * 16,), jnp.float32),
            pltpu.VMEM((CH,), jnp.int32),
            pltpu.VMEM((CH,), jnp.int32),
            pltpu.VMEM((CH,), jnp.int32),
            pltpu.VMEM((CH,), jnp.int32),
            pltpu.VMEM((CH,), jnp.int32),
            pltpu.VMEM((CH,), jnp.int32),
            pltpu.VMEM((CH,), jnp.int32),
            pltpu.VMEM((CH,), jnp.int32),
            pltpu.VMEM((TAIL,), jnp.int32),
            pltpu.VMEM((TAIL,), jnp.int32),
            pltpu.VMEM((TAIL, ROW), jnp.float32),
            pltpu.VMEM((TAIL, ROW), jnp.float32),
            pltpu.SemaphoreType.DMA,
            pltpu.SemaphoreType.DMA,
            pltpu.SemaphoreType.DMA,
            pltpu.SemaphoreType.DMA,
            pltpu.SemaphoreType.DMA,
            pltpu.SemaphoreType.DMA,
            pltpu.SemaphoreType.DMA,
            pltpu.SemaphoreType.DMA,
        ],
    )


_edge_l12 = _make_edge_kernel(3)
_edge_l3 = _make_edge_kernel(4)


_BLK = 1000
_GRID = N_NODES // _BLK


def _full(shape):
    return pl.BlockSpec(shape, lambda i: (0,) * len(shape))


def _rows(w):
    return pl.BlockSpec((_BLK, w), lambda i: (i, 0))


def _node_tail(h, As, Ad, E):
    a_s = jnp.dot(h, As, preferred_element_type=jnp.float32)
    a_d = jnp.dot(h, Ad, preferred_element_type=jnp.float32)
    w = jnp.exp(_leaky(a_s + a_d))
    wexp = jnp.dot(w, E, preferred_element_type=jnp.float32)
    ones = jnp.ones(a_s.shape, jnp.float32)
    z48 = jnp.zeros((a_s.shape[0], 48), jnp.float32)
    z56 = jnp.zeros((a_s.shape[0], 56), jnp.float32)
    z112 = jnp.zeros((a_s.shape[0], 112), jnp.float32)
    z8 = jnp.zeros(a_s.shape, jnp.float32)
    table = jnp.concatenate([h, ones, a_s, z48], axis=1)
    adt = jnp.concatenate([z8, a_d, z112], axis=1)
    selfmsg = jnp.concatenate([h * wexp, w, z56], axis=1)
    return table, adt, selfmsg


def _stage1_body(x, W, As, Ad, E, table, adt, selfmsg):
    h = jnp.dot(x[...], W[...], preferred_element_type=jnp.float32)
    t, a, s = _node_tail(h, As[...], Ad[...], E[...])
    table[...], adt[...], selfmsg[...] = t, a, s


def _stage2_body(p0, p1, sm, x, Wx, Wh, b, As, Ad, E,
                 table, adt, selfmsg):
    tot = p0[...] + p1[...] + sm[...]
    r = 1.0 / (tot[:, 64:72] + 1e-16)
    t = tot[:, :64] * jnp.dot(r, E[...],
                              preferred_element_type=jnp.float32) + b[...]
    h1 = jnp.where(t > 0, t, jnp.exp(t) - 1.0)
    h = (jnp.dot(x[...], Wx[...], preferred_element_type=jnp.float32)
         + jnp.dot(h1, Wh[...], preferred_element_type=jnp.float32))
    t_, a_, s_ = _node_tail(h, As[...], Ad[...], E[...])
    table[...], adt[...], selfmsg[...] = t_, a_, s_


def _stage3_body(p0, p1, sm, W, b, As, Ad, E8, E16,
                 taba, tabb, ada, adb, selfa, selfb):
    tot = p0[...] + p1[...] + sm[...]
    r = 1.0 / (tot[:, 64:72] + 1e-16)
    t = tot[:, :64] * jnp.dot(r, E8[...],
                              preferred_element_type=jnp.float32) + b[...]
    h2 = jnp.where(t > 0, t, jnp.exp(t) - 1.0)
    h3 = jnp.dot(h2, W[...], preferred_element_type=jnp.float32)
    a_s = jnp.dot(h3, As[...], preferred_element_type=jnp.float32)
    a_d = jnp.dot(h3, Ad[...], preferred_element_type=jnp.float32)
    w = jnp.exp(_leaky(a_s + a_d))
    wexp = jnp.dot(w, E16[...], preferred_element_type=jnp.float32)
    B = a_s.shape[0]
    ones = jnp.ones((B, 8), jnp.float32)
    z4 = jnp.zeros((B, 4), jnp.float32)
    z48 = jnp.zeros((B, 48), jnp.float32)
    z56 = jnp.zeros((B, 56), jnp.float32)
    hw = h3 * wexp
    z8 = jnp.zeros((B, 8), jnp.float32)
    z112 = jnp.zeros((B, 112), jnp.float32)
    taba[...] = jnp.concatenate([h3[:, :64], ones, a_s[:, :4], z4, z48], 1)
    tabb[...] = jnp.concatenate([h3[:, 64:], ones, a_s[:, 4:], z4, z48], 1)
    ada[...] = jnp.concatenate([z8, a_d[:, :4], z4, z112], 1)
    adb[...] = jnp.concatenate([z8, a_d[:, 4:], z4, z112], 1)
    selfa[...] = jnp.concatenate([hw[:, :64], w[:, :4], z4, z56], 1)
    selfb[...] = jnp.concatenate([hw[:, 64:], w[:, 4:], z4, z56], 1)


def _stage4_body(p0a, p1a, sma, p0b, p1b, smb, Ma, Mb, b, E4, out):
    tota = p0a[...] + p1a[...] + sma[...]
    totb = p0b[...] + p1b[...] + smb[...]
    ra = 1.0 / (tota[:, 64:68] + 1e-16)
    rb = 1.0 / (totb[:, 64:68] + 1e-16)
    va = tota[:, :64] * jnp.dot(ra, E4[...],
                                preferred_element_type=jnp.float32)
    vb = totb[:, :64] * jnp.dot(rb, E4[...],
                                preferred_element_type=jnp.float32)
    o = (jnp.dot(va, Ma[...], preferred_element_type=jnp.float32)
         + jnp.dot(vb, Mb[...], preferred_element_type=jnp.float32)
         + b[...])
    m = jnp.max(o, axis=1, keepdims=True)
    ls = jnp.log(jnp.sum(jnp.exp(o - m), axis=1, keepdims=True))
    out[...] = o - m - ls


def _scatter_mat(a, ch):
    return jnp.einsum("hc,hk->hck", a, jnp.eye(HEADS, dtype=a.dtype)
                      ).reshape(HEADS * ch, HEADS)


def kernel(x, edge_index, W1, a1s, a1d, b1, W2, a2s, a2d, b2,
           W3, a3s, a3d, b3):
    ei = edge_index.astype(jnp.int32)
    eis, eid = ei[0], ei[1]
    E8 = jnp.repeat(jnp.eye(HEADS, dtype=jnp.float32), 8, axis=1)
    E16 = jnp.repeat(jnp.eye(HEADS, dtype=jnp.float32), 16, axis=1)
    E4 = jnp.repeat(jnp.eye(4, dtype=jnp.float32), 16, axis=1)
    Ma = jnp.tile(jnp.eye(16, dtype=jnp.float32), (4, 1)) / HEADS
    As1, Ad1 = _scatter_mat(a1s[0], 8), _scatter_mat(a1d[0], 8)
    As2, Ad2 = _scatter_mat(a2s[0], 8), _scatter_mat(a2d[0], 8)
    As3, Ad3 = _scatter_mat(a3s[0], 16), _scatter_mat(a3d[0], 16)

    f32 = jnp.float32
    sds = jax.ShapeDtypeStruct
    out3 = [sds((N_NODES, 128), f32), sds((N_NODES, 128), f32),
            sds((N_NODES, 128), f32)]

    tab1, ad1, self1 = pl.pallas_call(
        _stage1_body, grid=(_GRID,),
        in_specs=[_rows(128), _full((128, 64)), _full((64, 8)),
                  _full((64, 8)), _full((8, 64))],
        out_specs=[_rows(128), _rows(128), _rows(128)],
        out_shape=out3,
    )(x, W1, As1, Ad1, E8)

    part1 = _edge_l12(tab1, ad1, eis, eid)

    tab2, ad2, self2 = pl.pallas_call(
        _stage2_body, grid=(_GRID,),
        in_specs=[_rows(128), _rows(128), _rows(128), _rows(128),
                  _full((128, 64)), _full((64, 64)), _full((1, 64)),
                  _full((64, 8)), _full((64, 8)), _full((8, 64))],
        out_specs=[_rows(128), _rows(128), _rows(128)],
        out_shape=out3,
    )(part1[:N_NODES], part1[N_NODES:], self1, x, W2[:128], W2[128:],
      b1.reshape(1, 64), As2, Ad2, E8)

    part2 = _edge_l12(tab2, ad2, eis, eid)

    tab3a, tab3b, ad3a, ad3b, self3a, self3b = pl.pallas_call(
        _stage3_body, grid=(_GRID,),
        in_specs=[_rows(128), _rows(128), _rows(128),
                  _full((64, 128)), _full((1, 64)),
                  _full((128, 8)), _full((128, 8)), _full((8, 64)),
                  _full((8, 128))],
        out_specs=[_rows(128), _rows(128), _rows(128), _rows(128),
                   _rows(128), _rows(128)],
        out_shape=[sds((N_NODES, 128), f32)] * 6,
    )(part2[:N_NODES], part2[N_NODES:], self2, W3, b2.reshape(1, 64),
      As3, Ad3, E8, E16)

    part3a = _edge_l3(tab3a, ad3a, eis, eid)
    part3b = _edge_l3(tab3b, ad3b, eis, eid)

    out = pl.pallas_call(
        _stage4_body, grid=(_GRID,),
        in_specs=[_rows(128), _rows(128), _rows(128),
                  _rows(128), _rows(128), _rows(128),
                  _full((64, 16)), _full((64, 16)), _full((1, 16)),
                  _full((4, 64))],
        out_specs=_rows(16),
        out_shape=sds((N_NODES, 16), f32),
    )(part3a[:N_NODES], part3a[N_NODES:], self3a,
      part3b[:N_NODES], part3b[N_NODES:], self3b,
      Ma, Ma, b3.reshape(1, 16), E4)

    return out
